# baseline (speedup 1.0000x reference)
import jax
import jax.numpy as jnp
from jax.experimental import pallas as pl
from jax.experimental.pallas import tpu as pltpu

NUM_EXPERTS = 64
HIDDEN = 1024
INTER = 512
TOKENS = 64
TOP_K = 8

EPB = 1
FC = INTER // 2


def _moe_body(x_ref, idx_ref, w_ref, g0_ref, g1_ref, u0_ref, u1_ref,
              dn0_ref, dn1_ref, out_ref):
    step = pl.program_id(0)
    x = x_ref[...]
    acc = jnp.zeros((TOKENS, HIDDEN), jnp.float32)
    for i in range(EPB):
        e = step * EPB + i
        hs = []
        for g_ref, u_ref in ((g0_ref, u0_ref), (g1_ref, u1_ref)):
            gate = jax.lax.dot_general(
                x, g_ref[i], (((1,), (1,)), ((), ())),
                preferred_element_type=jnp.float32)
            up = jax.lax.dot_general(
                x, u_ref[i], (((1,), (1,)), ((), ())),
                preferred_element_type=jnp.float32)
            hs.append(gate * jax.nn.sigmoid(gate) * up)
        h = jnp.concatenate(hs, axis=1)
        out0 = jax.lax.dot_general(
            h, dn0_ref[i], (((1,), (1,)), ((), ())),
            preferred_element_type=jnp.float32)
        out1 = jax.lax.dot_general(
            h, dn1_ref[i], (((1,), (1,)), ((), ())),
            preferred_element_type=jnp.float32)
        out_e = jnp.concatenate([out0, out1], axis=1)
        sel = (idx_ref[...] == e).astype(jnp.float32)
        combine = jnp.sum(sel * w_ref[...], axis=1)
        acc = acc + out_e * combine[:, None]

    @pl.when(step == 0)
    def _init():
        out_ref[...] = acc

    @pl.when(step > 0)
    def _accum():
        out_ref[...] += acc


def kernel(hidden_states, top_k_index, top_k_weights, gate_up_proj, down_proj):
    return pl.pallas_call(
        _moe_body,
        grid=(NUM_EXPERTS // EPB,),
        in_specs=[
            pl.BlockSpec((TOKENS, HIDDEN), lambda e: (0, 0)),
            pl.BlockSpec((TOKENS, TOP_K), lambda e: (0, 0)),
            pl.BlockSpec((TOKENS, TOP_K), lambda e: (0, 0)),
            pl.BlockSpec((EPB, FC, HIDDEN), lambda e: (e, 0, 0)),
            pl.BlockSpec((EPB, FC, HIDDEN), lambda e: (e, 1, 0)),
            pl.BlockSpec((EPB, FC, HIDDEN), lambda e: (e, 2, 0)),
            pl.BlockSpec((EPB, FC, HIDDEN), lambda e: (e, 3, 0)),
            pl.BlockSpec((EPB, HIDDEN // 2, INTER), lambda e: (e, 0, 0)),
            pl.BlockSpec((EPB, HIDDEN // 2, INTER), lambda e: (e, 1, 0)),
        ],
        out_specs=pl.BlockSpec((TOKENS, HIDDEN), lambda e: (0, 0)),
        out_shape=jax.ShapeDtypeStruct((TOKENS, HIDDEN), jnp.float32),
        compiler_params=pltpu.CompilerParams(
            dimension_semantics=("arbitrary",),
        ),
    )(hidden_states, top_k_index, top_k_weights,
      gate_up_proj, gate_up_proj, gate_up_proj, gate_up_proj,
      down_proj, down_proj)


# speedup vs baseline: 1.1033x; 1.1033x over previous
import jax
import jax.numpy as jnp
from jax.experimental import pallas as pl
from jax.experimental.pallas import tpu as pltpu

NUM_EXPERTS = 64
HIDDEN = 1024
INTER = 512
TOKENS = 64
TOP_K = 8

EPB = 2
FC = INTER // 2


def _moe_body(x_ref, idx_ref, w_ref, g0_ref, g1_ref, u0_ref, u1_ref,
              dn0_ref, dn1_ref, out_ref):
    step = pl.program_id(0)
    x = x_ref[...]
    acc = jnp.zeros((TOKENS, HIDDEN), jnp.float32)
    for i in range(EPB):
        e = step * EPB + i
        hs = []
        for g_ref, u_ref in ((g0_ref, u0_ref), (g1_ref, u1_ref)):
            gate = jax.lax.dot_general(
                x, g_ref[i], (((1,), (1,)), ((), ())),
                preferred_element_type=jnp.float32)
            up = jax.lax.dot_general(
                x, u_ref[i], (((1,), (1,)), ((), ())),
                preferred_element_type=jnp.float32)
            hs.append(gate * jax.nn.sigmoid(gate) * up)
        h = jnp.concatenate(hs, axis=1)
        out0 = jax.lax.dot_general(
            h, dn0_ref[i], (((1,), (1,)), ((), ())),
            preferred_element_type=jnp.float32)
        out1 = jax.lax.dot_general(
            h, dn1_ref[i], (((1,), (1,)), ((), ())),
            preferred_element_type=jnp.float32)
        out_e = jnp.concatenate([out0, out1], axis=1)
        sel = (idx_ref[...] == e).astype(jnp.float32)
        combine = jnp.sum(sel * w_ref[...], axis=1)
        acc = acc + out_e * combine[:, None]

    @pl.when(step == 0)
    def _init():
        out_ref[...] = acc

    @pl.when(step > 0)
    def _accum():
        out_ref[...] += acc


def kernel(hidden_states, top_k_index, top_k_weights, gate_up_proj, down_proj):
    return pl.pallas_call(
        _moe_body,
        grid=(NUM_EXPERTS // EPB,),
        in_specs=[
            pl.BlockSpec((TOKENS, HIDDEN), lambda e: (0, 0)),
            pl.BlockSpec((TOKENS, TOP_K), lambda e: (0, 0)),
            pl.BlockSpec((TOKENS, TOP_K), lambda e: (0, 0)),
            pl.BlockSpec((EPB, FC, HIDDEN), lambda e: (e, 0, 0)),
            pl.BlockSpec((EPB, FC, HIDDEN), lambda e: (e, 1, 0)),
            pl.BlockSpec((EPB, FC, HIDDEN), lambda e: (e, 2, 0)),
            pl.BlockSpec((EPB, FC, HIDDEN), lambda e: (e, 3, 0)),
            pl.BlockSpec((EPB, HIDDEN // 2, INTER), lambda e: (e, 0, 0)),
            pl.BlockSpec((EPB, HIDDEN // 2, INTER), lambda e: (e, 1, 0)),
        ],
        out_specs=pl.BlockSpec((TOKENS, HIDDEN), lambda e: (0, 0)),
        out_shape=jax.ShapeDtypeStruct((TOKENS, HIDDEN), jnp.float32),
        compiler_params=pltpu.CompilerParams(
            dimension_semantics=("arbitrary",),
        ),
    )(hidden_states, top_k_index, top_k_weights,
      gate_up_proj, gate_up_proj, gate_up_proj, gate_up_proj,
      down_proj, down_proj)
